# baseline (speedup 1.0000x reference)
import functools

import jax
import jax.numpy as jnp
from jax import lax
from jax.experimental import pallas as pl
from jax.experimental.pallas import tpu as pltpu
from jax.experimental.pallas import tpu_sc as plsc

N = 10000
E = 320000
D = 128
H = 256
HD = D // 2

L = 16
NC = 2
NS = 16
K = 128
NBUF = 2
CH_PER_TILE = 80
GROUPS = CH_PER_TILE // NBUF
NCH = NC * NS * CH_PER_TILE
E_PAD = NCH * K
NP = 10240
ROWS_PER_TILE = NP // NS


BE = 2560
NBLK_E = E // BE


def _edge_mlp_body(eft_ref, w1t_ref, b1_ref, w2t_ref, b2_ref, out_ref):
    h = jnp.dot(w1t_ref[...], eft_ref[...], preferred_element_type=jnp.float32)
    h = jnp.maximum(h + b1_ref[...], 0.0)
    z = jnp.dot(w2t_ref[...], h, preferred_element_type=jnp.float32)
    out_ref[...] = jax.nn.sigmoid(z + b2_ref[...])[None]


def _edge_mlp(eft, w1t, b1c, w2t, b2c):
    return pl.pallas_call(
        _edge_mlp_body,
        grid=(NBLK_E,),
        in_specs=[
            pl.BlockSpec((3, BE), lambda i: (0, i)),
            pl.BlockSpec((H, 3), lambda i: (0, 0)),
            pl.BlockSpec((H, 1), lambda i: (0, 0)),
            pl.BlockSpec((1, H), lambda i: (0, 0)),
            pl.BlockSpec((1, 1), lambda i: (0, 0)),
        ],
        out_specs=pl.BlockSpec((1, 1, BE), lambda i: (i, 0, 0)),
        out_shape=jax.ShapeDtypeStruct((NBLK_E, 1, BE), jnp.float32),
    )(eft, w1t, b1c, w2t, b2c)


@functools.lru_cache(maxsize=None)
def _make_sc_aggregate():
    mesh = plsc.VectorSubcoreMesh(
        core_axis_name="c", subcore_axis_name="s",
        num_cores=NC, num_subcores=NS,
    )
    return functools.partial(
        pl.kernel,
        out_type=jax.ShapeDtypeStruct((NC * NP, D), jnp.float32),
        mesh=mesh,
        scratch_types=[
            pltpu.VMEM((3, K), jnp.int32),
            pltpu.VMEM((K, D), jnp.float32),
            pltpu.VMEM_SHARED((NP, D), jnp.float32),
            pltpu.SemaphoreType.DMA,
        ],
        compiler_params=pltpu.CompilerParams(needs_layout_passes=False),
    )(_sc_aggregate_body)


def _sc_aggregate_body(xp, edata, zrows, agg2, ebuf, rows, aggsh, sem):
    c = lax.axis_index("c")
    s = lax.axis_index("s")
    cn = c * NP

    pltpu.sync_copy(zrows, aggsh.at[pl.ds(s * ROWS_PER_TILE, ROWS_PER_TILE)])
    plsc.subcore_barrier()

    def chunk_body(i, _):
        ch = (c * NS + s) * CH_PER_TILE + i
        pltpu.sync_copy(edata.at[ch], ebuf)

        pltpu.async_copy(xp.at[ebuf.at[0]], rows, sem).wait()

        def me(e, _):
            widx = jnp.full((L,), e, jnp.int32)
            wrow = jnp.full((L,), 2, jnp.int32)
            wv = plsc.bitcast(plsc.load_gather(ebuf, [wrow, widx]),
                              jnp.float32)
            for g in range(D // L):
                rows[e, pl.ds(g * L, L)] = rows[e, pl.ds(g * L, L)] * wv
            return 0
        lax.fori_loop(0, K, me, 0)

        pltpu.sync_copy(rows, aggsh.at[ebuf.at[1]], add=True)
        return 0

    lax.fori_loop(0, CH_PER_TILE, chunk_body, 0)
    plsc.subcore_barrier()
    pltpu.sync_copy(
        aggsh.at[pl.ds(s * ROWS_PER_TILE, ROWS_PER_TILE)],
        agg2.at[pl.ds(cn + s * ROWS_PER_TILE, ROWS_PER_TILE)],
    )


def _sc_aggregate(xp, edata, zrows):
    return _make_sc_aggregate()(xp, edata, zrows)


BN = 2000
NBLK_N = N // BN


def _node_body(x_ref, a0_ref, a1_ref, deg_ref, a_ref, b_ref,
               wd_ref, bm1_ref, wm2_ref, bm2_ref, g_ref, beta_ref, out_ref):
    x = x_ref[...]
    deg = deg_ref[...]
    rd = 1.0 / jnp.maximum(deg, 1.0)
    agg = (a0_ref[...] + a1_ref[...]) * rd
    h = (jnp.dot(x, a_ref[...], preferred_element_type=jnp.float32)
         + jnp.dot(agg, b_ref[...], preferred_element_type=jnp.float32)
         + deg * wd_ref[...]
         + bm1_ref[...])
    h = jnp.maximum(h, 0.0)
    y = x + jnp.dot(h, wm2_ref[...], preferred_element_type=jnp.float32) + bm2_ref[...]
    mean = jnp.mean(y, axis=-1, keepdims=True)
    yc = y - mean
    var = jnp.mean(yc * yc, axis=-1, keepdims=True)
    out_ref[...] = yc * lax.rsqrt(var + 1e-5) * g_ref[...] + beta_ref[...]


def _node_mlp(x, a0, a1, deg, a, b, wd, bm1, wm2, bm2, g, beta):
    full = lambda r, c: pl.BlockSpec((r, c), lambda i: (0, 0))
    return pl.pallas_call(
        _node_body,
        grid=(NBLK_N,),
        in_specs=[
            pl.BlockSpec((BN, D), lambda i: (i, 0)),
            pl.BlockSpec((BN, D), lambda i: (i, 0)),
            pl.BlockSpec((BN, D), lambda i: (i, 0)),
            pl.BlockSpec((BN, 1), lambda i: (i, 0)),
            full(D, H), full(D, H), full(1, H), full(1, H),
            full(H, D), full(1, D), full(1, D), full(1, D),
        ],
        out_specs=pl.BlockSpec((BN, D), lambda i: (i, 0)),
        out_shape=jax.ShapeDtypeStruct((N, D), jnp.float32),
    )(x, a0, a1, deg, a, b, wd, bm1, wm2, bm2, g, beta)


def kernel(x, edge_index, edge_feat, degrees, W1, b1, W2, b2,
           Wm1, bm1, Wm2, bm2, gamma, beta):
    src = edge_index[0]
    dst = edge_index[1]

    w = _edge_mlp(edge_feat.T, W1.T, b1[:, None], W2.T,
                  b2.reshape(1, 1)).reshape(E)

    pad = E_PAD - E
    dst_p = jnp.pad(dst, (0, pad)).reshape(NCH, K)
    src_p = jnp.pad(src, (0, pad)).reshape(NCH, K)
    w_p = jnp.pad(w, (0, pad)).reshape(NCH, K)
    edata = jnp.stack(
        [dst_p, src_p, w_p.view(jnp.int32)], axis=1)

    xp = jnp.pad(x, ((0, NP - N), (0, 0)))
    zrows = jnp.zeros((ROWS_PER_TILE, D), jnp.float32)

    agg2 = _sc_aggregate(xp, edata, zrows)

    out = _node_mlp(
        x, agg2[:N], agg2[NP:NP + N], degrees[:, None],
        Wm1[:D], Wm1[D:2 * D], Wm1[2 * D:2 * D + 1],
        bm1[None, :], Wm2, bm2[None, :], gamma[None, :], beta[None, :])
    return out


# speedup vs baseline: 1.7974x; 1.7974x over previous
import functools

import jax
import jax.numpy as jnp
from jax import lax
from jax.experimental import pallas as pl
from jax.experimental.pallas import tpu as pltpu
from jax.experimental.pallas import tpu_sc as plsc

N = 10000
E = 320000
D = 128
H = 256
HD = D // 2

L = 16
NC = 2
NS = 16
K = 128
CH_PER_TILE = -(-E // (NC * NS * K))
NCH = NC * NS * CH_PER_TILE
E_PAD = NCH * K
NP = 10240
ROWS_PER_TILE = NP // NS


BE = 2560
NBLK_E = E // BE


def _edge_mlp_body(eft_ref, w1t_ref, b1_ref, w2t_ref, b2_ref, out_ref):
    h = jnp.dot(w1t_ref[...], eft_ref[...], preferred_element_type=jnp.float32)
    h = jnp.maximum(h + b1_ref[...], 0.0)
    z = jnp.dot(w2t_ref[...], h, preferred_element_type=jnp.float32)
    out_ref[...] = jax.nn.sigmoid(z + b2_ref[...])[None]


def _edge_mlp(eft, w1t, b1c, w2t, b2c):
    return pl.pallas_call(
        _edge_mlp_body,
        grid=(NBLK_E,),
        in_specs=[
            pl.BlockSpec((3, BE), lambda i: (0, i)),
            pl.BlockSpec((H, 3), lambda i: (0, 0)),
            pl.BlockSpec((H, 1), lambda i: (0, 0)),
            pl.BlockSpec((1, H), lambda i: (0, 0)),
            pl.BlockSpec((1, 1), lambda i: (0, 0)),
        ],
        out_specs=pl.BlockSpec((1, 1, BE), lambda i: (i, 0, 0)),
        out_shape=jax.ShapeDtypeStruct((NBLK_E, 1, BE), jnp.float32),
    )(eft, w1t, b1c, w2t, b2c)


@functools.lru_cache(maxsize=None)
def _make_sc_aggregate():
    mesh = plsc.VectorSubcoreMesh(
        core_axis_name="c", subcore_axis_name="s",
        num_cores=NC, num_subcores=NS,
    )
    return functools.partial(
        pl.kernel,
        out_type=jax.ShapeDtypeStruct((NC * NP, D), jnp.float32),
        mesh=mesh,
        scratch_types=[
            pltpu.VMEM((3, K), jnp.int32),
            pltpu.VMEM((K, D), jnp.float32),
            pltpu.VMEM_SHARED((NP, D), jnp.float32),
            pltpu.SemaphoreType.DMA,
        ],
        compiler_params=pltpu.CompilerParams(needs_layout_passes=False),
    )(_sc_aggregate_body)


def _sc_aggregate_body(xp, edata, zrows, agg2, ebuf, rows, aggsh, sem):
    c = lax.axis_index("c")
    s = lax.axis_index("s")
    cn = c * NP

    pltpu.sync_copy(zrows, aggsh.at[pl.ds(s * ROWS_PER_TILE, ROWS_PER_TILE)])
    plsc.subcore_barrier()

    def chunk_body(i, _):
        ch = (c * NS + s) * CH_PER_TILE + i
        pltpu.sync_copy(edata.at[ch], ebuf)

        pltpu.async_copy(xp.at[ebuf.at[0]], rows, sem).wait()

        def me(e, _):
            widx = jnp.full((L,), e, jnp.int32)
            wrow = jnp.full((L,), 2, jnp.int32)
            wv = plsc.bitcast(plsc.load_gather(ebuf, [wrow, widx]),
                              jnp.float32)
            for g in range(D // L):
                rows[e, pl.ds(g * L, L)] = rows[e, pl.ds(g * L, L)] * wv
            return 0
        lax.fori_loop(0, K, me, 0)

        pltpu.sync_copy(rows, aggsh.at[ebuf.at[1]], add=True)
        return 0

    lax.fori_loop(0, CH_PER_TILE, chunk_body, 0)
    plsc.subcore_barrier()
    pltpu.sync_copy(
        aggsh.at[pl.ds(s * ROWS_PER_TILE, ROWS_PER_TILE)],
        agg2.at[pl.ds(cn + s * ROWS_PER_TILE, ROWS_PER_TILE)],
    )


def _sc_aggregate(xp, edata, zrows):
    return _make_sc_aggregate()(xp, edata, zrows)


BN = 2000
NBLK_N = N // BN


def _node_body(x_ref, a0_ref, a1_ref, deg_ref, a_ref, b_ref,
               wd_ref, bm1_ref, wm2_ref, bm2_ref, g_ref, beta_ref, out_ref):
    x = x_ref[...]
    deg = deg_ref[...]
    rd = 1.0 / jnp.maximum(deg, 1.0)
    agg = (a0_ref[...] + a1_ref[...]) * rd
    h = (jnp.dot(x, a_ref[...], preferred_element_type=jnp.float32)
         + jnp.dot(agg, b_ref[...], preferred_element_type=jnp.float32)
         + deg * wd_ref[...]
         + bm1_ref[...])
    h = jnp.maximum(h, 0.0)
    y = x + jnp.dot(h, wm2_ref[...], preferred_element_type=jnp.float32) + bm2_ref[...]
    mean = jnp.mean(y, axis=-1, keepdims=True)
    yc = y - mean
    var = jnp.mean(yc * yc, axis=-1, keepdims=True)
    out_ref[...] = yc * lax.rsqrt(var + 1e-5) * g_ref[...] + beta_ref[...]


def _node_mlp(x, a0, a1, deg, a, b, wd, bm1, wm2, bm2, g, beta):
    full = lambda r, c: pl.BlockSpec((r, c), lambda i: (0, 0))
    return pl.pallas_call(
        _node_body,
        grid=(NBLK_N,),
        in_specs=[
            pl.BlockSpec((BN, D), lambda i: (i, 0)),
            pl.BlockSpec((BN, D), lambda i: (i, 0)),
            pl.BlockSpec((BN, D), lambda i: (i, 0)),
            pl.BlockSpec((BN, 1), lambda i: (i, 0)),
            full(D, H), full(D, H), full(1, H), full(1, H),
            full(H, D), full(1, D), full(1, D), full(1, D),
        ],
        out_specs=pl.BlockSpec((BN, D), lambda i: (i, 0)),
        out_shape=jax.ShapeDtypeStruct((N, D), jnp.float32),
    )(x, a0, a1, deg, a, b, wd, bm1, wm2, bm2, g, beta)


def kernel(x, edge_index, edge_feat, degrees, W1, b1, W2, b2,
           Wm1, bm1, Wm2, bm2, gamma, beta):
    src = edge_index[0]
    dst = edge_index[1]

    w = _edge_mlp(edge_feat.T, W1.T, b1[:, None], W2.T,
                  b2.reshape(1, 1)).reshape(E)

    pad = E_PAD - E
    fill = jnp.arange(pad, dtype=jnp.int32) % N
    dst_p = jnp.concatenate([dst, fill]).reshape(NCH, K)
    src_p = jnp.concatenate([src, fill]).reshape(NCH, K)
    w_p = jnp.pad(w, (0, pad)).reshape(NCH, K)
    edata = jnp.stack(
        [dst_p, src_p, w_p.view(jnp.int32)], axis=1)

    xp = jnp.pad(x, ((0, NP - N), (0, 0)))
    zrows = jnp.zeros((ROWS_PER_TILE, D), jnp.float32)

    agg2 = _sc_aggregate(xp, edata, zrows)

    out = _node_mlp(
        x, agg2[:N], agg2[NP:NP + N], degrees[:, None],
        Wm1[:D], Wm1[D:2 * D], Wm1[2 * D:2 * D + 1],
        bm1[None, :], Wm2, bm2[None, :], gamma[None, :], beta[None, :])
    return out


# speedup vs baseline: 2.5984x; 1.4457x over previous
import functools

import jax
import jax.numpy as jnp
from jax import lax
from jax.experimental import pallas as pl
from jax.experimental.pallas import tpu as pltpu
from jax.experimental.pallas import tpu_sc as plsc

N = 10000
E = 320000
D = 128
H = 256
HD = D // 2

L = 16
NC = 2
NS = 16
K = 128
NBUF = 2
CH_PER_TILE = 80
GROUPS = CH_PER_TILE // NBUF
NCH = NC * NS * CH_PER_TILE
E_PAD = NCH * K
NP = 10240
ROWS_PER_TILE = NP // NS


BE = 2560
NBLK_E = E // BE


def _edge_mlp_body(eft_ref, w1t_ref, b1_ref, w2t_ref, b2_ref, out_ref):
    h = jnp.dot(w1t_ref[...], eft_ref[...], preferred_element_type=jnp.float32)
    h = jnp.maximum(h + b1_ref[...], 0.0)
    z = jnp.dot(w2t_ref[...], h, preferred_element_type=jnp.float32)
    out_ref[...] = jax.nn.sigmoid(z + b2_ref[...])[None]


def _edge_mlp(eft, w1t, b1c, w2t, b2c):
    return pl.pallas_call(
        _edge_mlp_body,
        grid=(NBLK_E,),
        in_specs=[
            pl.BlockSpec((3, BE), lambda i: (0, i)),
            pl.BlockSpec((H, 3), lambda i: (0, 0)),
            pl.BlockSpec((H, 1), lambda i: (0, 0)),
            pl.BlockSpec((1, H), lambda i: (0, 0)),
            pl.BlockSpec((1, 1), lambda i: (0, 0)),
        ],
        out_specs=pl.BlockSpec((1, 1, BE), lambda i: (i, 0, 0)),
        out_shape=jax.ShapeDtypeStruct((NBLK_E, 1, BE), jnp.float32),
    )(eft, w1t, b1c, w2t, b2c)


@functools.lru_cache(maxsize=None)
def _make_sc_aggregate():
    mesh = plsc.VectorSubcoreMesh(
        core_axis_name="c", subcore_axis_name="s",
        num_cores=NC, num_subcores=NS,
    )
    return functools.partial(
        pl.kernel,
        out_type=jax.ShapeDtypeStruct((NC * NP, D), jnp.float32),
        mesh=mesh,
        scratch_types=[
            pltpu.VMEM((3, K), jnp.int32),
            pltpu.VMEM((3, K), jnp.int32),
            pltpu.VMEM((K, D), jnp.float32),
            pltpu.VMEM((K, D), jnp.float32),
            pltpu.VMEM_SHARED((NP, D), jnp.float32),
            pltpu.SemaphoreType.DMA,
            pltpu.SemaphoreType.DMA,
            pltpu.SemaphoreType.DMA,
            pltpu.SemaphoreType.DMA,
        ],
        compiler_params=pltpu.CompilerParams(needs_layout_passes=False),
    )(_sc_aggregate_body)


def _sc_aggregate_body(xp, edata, zrows, agg2, ebuf0, ebuf1, rows0, rows1,
                       aggsh, esem0, esem1, gsem0, gsem1):
    c = lax.axis_index("c")
    s = lax.axis_index("s")
    cn = c * NP
    base = (c * NS + s) * CH_PER_TILE
    ebufs = (ebuf0, ebuf1)
    rowss = (rows0, rows1)
    esems = (esem0, esem1)
    gsems = (gsem0, gsem1)

    pltpu.sync_copy(zrows, aggsh.at[pl.ds(s * ROWS_PER_TILE, ROWS_PER_TILE)])
    plsc.subcore_barrier()

    pltpu.async_copy(edata.at[base], ebuf0, esem0).wait()
    pltpu.async_copy(xp.at[ebuf0.at[0]], rows0, gsem0)

    def chunk_pair(t, _):
        for b in range(2):
            i = 2 * t + b
            o = 1 - b
            nxt = base + jnp.minimum(i + 1, CH_PER_TILE - 1)
            pltpu.async_copy(edata.at[nxt], ebufs[o], esems[o]).wait()
            pltpu.async_copy(xp.at[ebufs[o].at[0]], rowss[o], gsems[o])
            pltpu.make_async_copy(xp.at[ebufs[b].at[0]], rowss[b],
                                  gsems[b]).wait()
            eb = ebufs[b]
            rb = rowss[b]
            wrow = jnp.full((L,), 2, jnp.int32)

            @plsc.parallel_loop(0, K, 1, unroll=4)
            def me(e):
                widx = jnp.full((L,), e, jnp.int32)
                wv = plsc.bitcast(plsc.load_gather(eb, [wrow, widx]),
                                  jnp.float32)
                for g in range(D // L):
                    rb[e, pl.ds(g * L, L)] = rb[e, pl.ds(g * L, L)] * wv

            pltpu.sync_copy(rb, aggsh.at[eb.at[1]], add=True)
        return 0

    lax.fori_loop(0, CH_PER_TILE // 2, chunk_pair, 0)
    pltpu.make_async_copy(xp.at[ebuf0.at[0]], rows0, gsem0).wait()
    plsc.subcore_barrier()
    pltpu.sync_copy(
        aggsh.at[pl.ds(s * ROWS_PER_TILE, ROWS_PER_TILE)],
        agg2.at[pl.ds(cn + s * ROWS_PER_TILE, ROWS_PER_TILE)],
    )


def _sc_aggregate(xp, edata, zrows):
    return _make_sc_aggregate()(xp, edata, zrows)


BN = 2000
NBLK_N = N // BN


def _node_body(x_ref, a0_ref, a1_ref, deg_ref, a_ref, b_ref,
               wd_ref, bm1_ref, wm2_ref, bm2_ref, g_ref, beta_ref, out_ref):
    x = x_ref[...]
    deg = deg_ref[...]
    rd = 1.0 / jnp.maximum(deg, 1.0)
    agg = (a0_ref[...] + a1_ref[...]) * rd
    h = (jnp.dot(x, a_ref[...], preferred_element_type=jnp.float32)
         + jnp.dot(agg, b_ref[...], preferred_element_type=jnp.float32)
         + deg * wd_ref[...]
         + bm1_ref[...])
    h = jnp.maximum(h, 0.0)
    y = x + jnp.dot(h, wm2_ref[...], preferred_element_type=jnp.float32) + bm2_ref[...]
    mean = jnp.mean(y, axis=-1, keepdims=True)
    yc = y - mean
    var = jnp.mean(yc * yc, axis=-1, keepdims=True)
    out_ref[...] = yc * lax.rsqrt(var + 1e-5) * g_ref[...] + beta_ref[...]


def _node_mlp(x, a0, a1, deg, a, b, wd, bm1, wm2, bm2, g, beta):
    full = lambda r, c: pl.BlockSpec((r, c), lambda i: (0, 0))
    return pl.pallas_call(
        _node_body,
        grid=(NBLK_N,),
        in_specs=[
            pl.BlockSpec((BN, D), lambda i: (i, 0)),
            pl.BlockSpec((BN, D), lambda i: (i, 0)),
            pl.BlockSpec((BN, D), lambda i: (i, 0)),
            pl.BlockSpec((BN, 1), lambda i: (i, 0)),
            full(D, H), full(D, H), full(1, H), full(1, H),
            full(H, D), full(1, D), full(1, D), full(1, D),
        ],
        out_specs=pl.BlockSpec((BN, D), lambda i: (i, 0)),
        out_shape=jax.ShapeDtypeStruct((N, D), jnp.float32),
    )(x, a0, a1, deg, a, b, wd, bm1, wm2, bm2, g, beta)


def kernel(x, edge_index, edge_feat, degrees, W1, b1, W2, b2,
           Wm1, bm1, Wm2, bm2, gamma, beta):
    src = edge_index[0]
    dst = edge_index[1]

    w = _edge_mlp(edge_feat.T, W1.T, b1[:, None], W2.T,
                  b2.reshape(1, 1)).reshape(E)

    pad = E_PAD - E
    fill = jnp.arange(pad, dtype=jnp.int32) % N
    dst_p = jnp.concatenate([dst, fill]).reshape(NCH, K)
    src_p = jnp.concatenate([src, fill]).reshape(NCH, K)
    w_p = jnp.pad(w, (0, pad)).reshape(NCH, K)
    edata = jnp.stack(
        [dst_p, src_p, w_p.view(jnp.int32)], axis=1)

    xp = jnp.pad(x, ((0, NP - N), (0, 0)))
    zrows = jnp.zeros((ROWS_PER_TILE, D), jnp.float32)

    agg2 = _sc_aggregate(xp, edata, zrows)

    out = _node_mlp(
        x, agg2[:N], agg2[NP:NP + N], degrees[:, None],
        Wm1[:D], Wm1[D:2 * D], Wm1[2 * D:2 * D + 1],
        bm1[None, :], Wm2, bm2[None, :], gamma[None, :], beta[None, :])
    return out
